# Initial kernel scaffold; baseline (speedup 1.0000x reference)
import jax
import jax.numpy as jnp
from jax.experimental import pallas as pl


def kernel(S_hat, edges_s, edges_t, W_self, W_nbr):
    raise NotImplementedError("write your pallas kernel here")


# speedup vs baseline: 2.2068x; 2.2068x over previous
import functools

import jax
import jax.numpy as jnp
from jax.experimental import pallas as pl
from jax.experimental.pallas import tpu as pltpu

K = 10
NUM_STEPS = 2
KP = K + 1
PAD = 16


def _topk_body(s_ref, shk_ref, idx_ref, *, n1):
    vals = s_ref[...]
    r = vals.shape[0]
    col = jax.lax.broadcasted_iota(jnp.int32, vals.shape, 1)
    neg = jnp.float32(-1e30)
    vcols = []
    icols = []
    for _ in range(KP):
        m = jnp.max(vals, axis=1)
        eq = vals == m[:, None]
        idx = jnp.min(jnp.where(eq, col, n1), axis=1)
        vcols.append(m[:, None])
        icols.append(idx[:, None])
        vals = jnp.where(col == idx[:, None], neg, vals)
    vcols.append(jnp.full((r, PAD - KP), neg, jnp.float32))
    icols.append(jnp.zeros((r, PAD - KP), jnp.int32))
    shk_ref[...] = jnp.concatenate(vcols, axis=1) * jnp.float32(NUM_STEPS)
    idx_ref[...] = jnp.concatenate(icols, axis=1)


def _topk(s_hat, block_rows):
    n0, n1 = s_hat.shape
    grid = n0 // block_rows
    return pl.pallas_call(
        functools.partial(_topk_body, n1=n1),
        grid=(grid,),
        in_specs=[pl.BlockSpec((block_rows, n1), lambda i: (i, 0))],
        out_specs=[
            pl.BlockSpec((block_rows, PAD), lambda i: (i, 0)),
            pl.BlockSpec((block_rows, PAD), lambda i: (i, 0)),
        ],
        out_shape=[
            jax.ShapeDtypeStruct((n0, PAD), jnp.float32),
            jax.ShapeDtypeStruct((n0, PAD), jnp.int32),
        ],
    )(s_hat)


def _psi2(r, edges, w_self, w_nbr):
    src = edges[0]
    dst = edges[1]
    agg = jnp.zeros_like(r).at[dst].add(r[src])
    return jax.nn.relu(r @ w_self + agg @ w_nbr)


def _l2norm(x):
    return x / jnp.clip(jnp.linalg.norm(x, axis=-1, keepdims=True), 1e-12, None)


def kernel(S_hat, edges_s, edges_t, W_self, W_nbr):
    n0, n1 = S_hat.shape
    rnd_dim = W_self.shape[0]
    block_rows = 200 if n0 % 200 == 0 else 8

    shk_pad, idx_pad = _topk(S_hat, block_rows)
    s_hat_k = shk_pad[:, :KP]
    knn_idx0 = idx_pad[:, :KP]
    S = jax.nn.softmax(s_hat_k, axis=1)

    row_flat = jnp.repeat(jnp.arange(n0, dtype=jnp.int32), KP)
    col_flat = knn_idx0.reshape(-1)
    rkey = jax.random.key(42)
    for step in range(NUM_STEPS):
        r_s = jax.random.normal(jax.random.fold_in(rkey, step), (n0, rnd_dim),
                                dtype=S_hat.dtype)
        vals = S.reshape(-1)
        r_t = jnp.zeros((n1, rnd_dim), dtype=S_hat.dtype).at[col_flat].add(
            vals[:, None] * r_s[row_flat])
        o_s = _psi2(r_s, edges_s, W_self, W_nbr)
        o_t = _psi2(r_t, edges_t, W_self, W_nbr)
        o_s = _l2norm(o_s)
        o_t = _l2norm(o_t)
        sim = jnp.sum(o_s[:, None, :] * o_t[knn_idx0], axis=-1)
        s_hat_k = s_hat_k + sim
        S = jax.nn.softmax(s_hat_k, axis=1)
    return S


# speedup vs baseline: 2.8395x; 1.2867x over previous
import functools

import jax
import jax.numpy as jnp
from jax import lax
from jax.experimental import pallas as pl
from jax.experimental.pallas import tpu as pltpu
from jax.experimental.pallas import tpu_sc as plsc

K = 10
NUM_STEPS = 2
KP = K + 1
PAD = 16

SC_CORES = 2
SC_TILES = 16
NW = SC_CORES * SC_TILES


def _edge_agg_body(r_hbm, src_hbm, dst_hbm, zeros_hbm, out_hbm,
                   idx_s_v, idx_d_v, rows_v, acc_sh, sem):
    c = lax.axis_index("c")
    s = lax.axis_index("s")
    w = c * SC_TILES + s
    n = acc_sh.shape[0]
    rpt = (n // SC_TILES) // 8 * 8
    last = n - rpt * (SC_TILES - 1)

    @pl.when(s < SC_TILES - 1)
    def _():
        pltpu.sync_copy(zeros_hbm.at[pl.ds(0, rpt)],
                        acc_sh.at[pl.ds(s * rpt, rpt)])

    @pl.when(s == SC_TILES - 1)
    def _():
        pltpu.sync_copy(zeros_hbm, acc_sh.at[pl.ds((SC_TILES - 1) * rpt, last)])

    pltpu.sync_copy(src_hbm.at[w], idx_s_v)
    pltpu.sync_copy(dst_hbm.at[w], idx_d_v)
    plsc.subcore_barrier()
    nc = idx_s_v.shape[0]

    def body(j, carry):
        pltpu.async_copy(r_hbm.at[idx_s_v.at[j]], rows_v, sem).wait()
        pltpu.sync_copy(rows_v, acc_sh.at[idx_d_v.at[j]], add=True)
        return carry

    lax.fori_loop(0, nc, body, 0, unroll=False)
    plsc.subcore_barrier()

    @pl.when(s < SC_TILES - 1)
    def _():
        pltpu.sync_copy(acc_sh.at[pl.ds(s * rpt, rpt)],
                        out_hbm.at[c, pl.ds(s * rpt, rpt)])

    @pl.when(s == SC_TILES - 1)
    def _():
        pltpu.sync_copy(acc_sh.at[pl.ds((SC_TILES - 1) * rpt, last)],
                        out_hbm.at[c, pl.ds((SC_TILES - 1) * rpt, last)])


def _edge_agg(r, edges, chunk=125):
    n, d = r.shape
    e = edges.shape[1]
    nc = e // (NW * chunk)
    assert nc * NW * chunk == e, (e, chunk)
    src = edges[0].reshape(NW, nc, chunk)
    dst = edges[1].reshape(NW, nc, chunk)
    rpt = (n // SC_TILES) // 8 * 8
    zeros = jnp.zeros((n - rpt * (SC_TILES - 1), d), jnp.float32)
    mesh = plsc.VectorSubcoreMesh(core_axis_name="c", subcore_axis_name="s")
    f = pl.kernel(
        _edge_agg_body,
        out_type=jax.ShapeDtypeStruct((SC_CORES, n, d), jnp.float32),
        mesh=mesh,
        scratch_types=[
            pltpu.VMEM((nc, chunk), jnp.int32),
            pltpu.VMEM((nc, chunk), jnp.int32),
            pltpu.VMEM((chunk, d), jnp.float32),
            pltpu.VMEM_SHARED((n, d), jnp.float32),
            pltpu.SemaphoreType.DMA,
        ],
    )
    return f(r, src, dst, zeros)


def _topk_body(s_ref, shk_ref, idx_ref, *, n1):
    vals = s_ref[...]
    r = vals.shape[0]
    col = jax.lax.broadcasted_iota(jnp.int32, vals.shape, 1)
    neg = jnp.float32(-1e30)
    vcols = []
    icols = []
    for _ in range(KP):
        m = jnp.max(vals, axis=1)
        eq = vals == m[:, None]
        idx = jnp.min(jnp.where(eq, col, n1), axis=1)
        vcols.append(m[:, None])
        icols.append(idx[:, None])
        vals = jnp.where(col == idx[:, None], neg, vals)
    vcols.append(jnp.full((r, PAD - KP), neg, jnp.float32))
    icols.append(jnp.zeros((r, PAD - KP), jnp.int32))
    shk_ref[...] = jnp.concatenate(vcols, axis=1) * jnp.float32(NUM_STEPS)
    idx_ref[...] = jnp.concatenate(icols, axis=1)


def _topk(s_hat, block_rows):
    n0, n1 = s_hat.shape
    grid = n0 // block_rows
    return pl.pallas_call(
        functools.partial(_topk_body, n1=n1),
        grid=(grid,),
        in_specs=[pl.BlockSpec((block_rows, n1), lambda i: (i, 0))],
        out_specs=[
            pl.BlockSpec((block_rows, PAD), lambda i: (i, 0)),
            pl.BlockSpec((block_rows, PAD), lambda i: (i, 0)),
        ],
        out_shape=[
            jax.ShapeDtypeStruct((n0, PAD), jnp.float32),
            jax.ShapeDtypeStruct((n0, PAD), jnp.int32),
        ],
    )(s_hat)


def _psi2(r, edges, w_self, w_nbr):
    ap = _edge_agg(r, edges)
    agg = ap[0] + ap[1]
    return jax.nn.relu(r @ w_self + agg @ w_nbr)


def _l2norm(x):
    return x / jnp.clip(jnp.linalg.norm(x, axis=-1, keepdims=True), 1e-12, None)


def kernel(S_hat, edges_s, edges_t, W_self, W_nbr):
    n0, n1 = S_hat.shape
    rnd_dim = W_self.shape[0]
    block_rows = 200 if n0 % 200 == 0 else 8

    shk_pad, idx_pad = _topk(S_hat, block_rows)
    s_hat_k = shk_pad[:, :KP]
    knn_idx0 = idx_pad[:, :KP]
    S = jax.nn.softmax(s_hat_k, axis=1)

    row_flat = jnp.repeat(jnp.arange(n0, dtype=jnp.int32), KP)
    col_flat = knn_idx0.reshape(-1)
    rkey = jax.random.key(42)
    for step in range(NUM_STEPS):
        r_s = jax.random.normal(jax.random.fold_in(rkey, step), (n0, rnd_dim),
                                dtype=S_hat.dtype)
        vals = S.reshape(-1)
        r_t = jnp.zeros((n1, rnd_dim), dtype=S_hat.dtype).at[col_flat].add(
            vals[:, None] * r_s[row_flat])
        o_s = _psi2(r_s, edges_s, W_self, W_nbr)
        o_t = _psi2(r_t, edges_t, W_self, W_nbr)
        o_s = _l2norm(o_s)
        o_t = _l2norm(o_t)
        sim = jnp.sum(o_s[:, None, :] * o_t[knn_idx0], axis=-1)
        s_hat_k = s_hat_k + sim
        S = jax.nn.softmax(s_hat_k, axis=1)
    return S
